# baseline (speedup 1.0000x reference)
import jax
import jax.numpy as jnp
from jax import lax
from jax.experimental import pallas as pl
from jax.experimental.pallas import tpu as pltpu
from jax.experimental.pallas import tpu_sc as plsc

_TOKENS = 32768
_DIM = 2048
_NE = 64
_K = 8
_BT = 2048
_NW = 32
_SPAN = 256
_CUT = 24576


def _score_body(x_ref, w_ref, s_ref):
    logits = lax.dot_general(
        x_ref[...], w_ref[...], (((1,), (1,)), ((), ())),
        preferred_element_type=jnp.float32,
    )
    m = jnp.max(logits, axis=1, keepdims=True)
    e = jnp.exp(logits - m)
    s = e / jnp.sum(e, axis=1, keepdims=True)
    parts = []
    for j in range(_BT // _SPAN):
        a = s[j * _SPAN: j * _SPAN + 128]
        b = s[j * _SPAN + 128: (j + 1) * _SPAN]
        parts.append(jnp.concatenate([a, b], axis=1))
    s_ref[...] = jnp.concatenate(parts, axis=0)


def _scores(x, peso, blk0, nblk):
    return pl.pallas_call(
        _score_body,
        grid=(nblk,),
        in_specs=[
            pl.BlockSpec((_BT, _DIM), lambda i: (blk0 + i, 0)),
            pl.BlockSpec((_NE, _DIM), lambda i: (0, 0)),
        ],
        out_specs=pl.BlockSpec((_BT // 2, 2 * _NE), lambda i: (i, 0)),
        out_shape=jax.ShapeDtypeStruct((nblk * _BT // 2, 2 * _NE), jnp.float32),
        compiler_params=pltpu.CompilerParams(
            dimension_semantics=("arbitrary",),
        ),
    )(x, peso)


def _merge(aK, aV, bK, bV):
    brK = lax.rev(bK, (0,))
    brV = lax.rev(bV, (0,))
    m = aK >= brK
    K = jnp.where(m, aK, brK)
    V = jnp.where(m, aV, brV)
    return plsc.sort_key_val(K, V, descending=True)


def _topk_sc(scores_flat, ntok):
    rpw = ntok // _NW

    def body(s_hbm, pesos_hbm, idx_hbm, s_v, p_v, i_v):
        wid = lax.axis_index("s") * 2 + lax.axis_index("c")
        base = wid * rpw
        pltpu.sync_copy(s_hbm.at[pl.ds(base * _NE, rpw * _NE)], s_v)

        lane = lax.iota(jnp.int32, 16)
        mask8 = lane < 8
        v0 = lane
        v1 = lane + 16
        v2 = lane + 32
        v3 = lane + 48
        lane128 = lane * 128

        @plsc.parallel_loop(0, rpw, unroll=8)
        def _row(r):
            off = (r // _SPAN) * (_SPAN * _NE) \
                + (r % 128) * 128 + ((r // 128) % 2) * _NE
            k0, i0 = plsc.sort_key_val(s_v[pl.ds(off, 16)], v0,
                                       descending=True)
            k1, i1 = plsc.sort_key_val(s_v[pl.ds(off + 16, 16)], v1,
                                       descending=True)
            k2, i2 = plsc.sort_key_val(s_v[pl.ds(off + 32, 16)], v2,
                                       descending=True)
            k3, i3 = plsc.sort_key_val(s_v[pl.ds(off + 48, 16)], v3,
                                       descending=True)
            ka, ia = _merge(k0, i0, k1, i1)
            kb, ib = _merge(k2, i2, k3, i3)
            kf, jf = _merge(ka, ia, kb, ib)
            pos = (r // 128) * 1024 + (r % 128) + lane128
            plsc.store_scatter(p_v, [pos], kf, mask=mask8)
            plsc.store_scatter(i_v, [pos], jf, mask=mask8)

        pltpu.sync_copy(p_v, pesos_hbm.at[pl.ds(base * _K, rpw * _K)])
        pltpu.sync_copy(i_v, idx_hbm.at[pl.ds(base * _K, rpw * _K)])

    mesh = plsc.VectorSubcoreMesh(core_axis_name="c", subcore_axis_name="s")
    f = pl.kernel(
        body,
        out_type=[
            jax.ShapeDtypeStruct((ntok * _K,), jnp.float32),
            jax.ShapeDtypeStruct((ntok * _K,), jnp.int32),
        ],
        mesh=mesh,
        scratch_types=[
            pltpu.VMEM((rpw * _NE,), jnp.float32),
            pltpu.VMEM((rpw * _K,), jnp.float32),
            pltpu.VMEM((rpw * _K,), jnp.int32),
        ],
        compiler_params=pltpu.CompilerParams(needs_layout_passes=False),
    )
    return f(scores_flat)


def _pack_body(pf_ref, if_ref, p_ref, i_ref):
    a3 = pf_ref[...].reshape(16, 8, 128)
    b3 = if_ref[...].reshape(16, 8, 128)
    for j in range(16):
        p_ref[:, pl.ds(128 * j, 128)] = a3[j]
        i_ref[:, pl.ds(128 * j, 128)] = b3[j]


def _pack_body2(pf_ref, if_ref, pa_ref, ia_ref, p_ref, i_ref):
    del pa_ref, ia_ref
    _pack_body(pf_ref, if_ref, p_ref, i_ref)


def _pack0(p_flat, i_flat):
    n = _CUT // 2048
    return pl.pallas_call(
        _pack_body,
        grid=(n,),
        in_specs=[
            pl.BlockSpec((128, 128), lambda i: (i, 0)),
            pl.BlockSpec((128, 128), lambda i: (i, 0)),
        ],
        out_specs=[
            pl.BlockSpec((_K, 2048), lambda i: (0, i)),
            pl.BlockSpec((_K, 2048), lambda i: (0, i)),
        ],
        out_shape=[
            jax.ShapeDtypeStruct((_K, _TOKENS), jnp.float32),
            jax.ShapeDtypeStruct((_K, _TOKENS), jnp.int32),
        ],
        compiler_params=pltpu.CompilerParams(
            dimension_semantics=("arbitrary",),
        ),
    )(p_flat.reshape(-1, 128), i_flat.reshape(-1, 128))


def _pack1(p_flat, i_flat, p_acc, i_acc):
    base = _CUT // 2048
    n = (_TOKENS - _CUT) // 2048
    return pl.pallas_call(
        _pack_body2,
        grid=(n,),
        in_specs=[
            pl.BlockSpec((128, 128), lambda i: (i, 0)),
            pl.BlockSpec((128, 128), lambda i: (i, 0)),
            pl.BlockSpec(memory_space=pl.ANY),
            pl.BlockSpec(memory_space=pl.ANY),
        ],
        out_specs=[
            pl.BlockSpec((_K, 2048), lambda i: (0, base + i)),
            pl.BlockSpec((_K, 2048), lambda i: (0, base + i)),
        ],
        out_shape=[
            jax.ShapeDtypeStruct((_K, _TOKENS), jnp.float32),
            jax.ShapeDtypeStruct((_K, _TOKENS), jnp.int32),
        ],
        input_output_aliases={2: 0, 3: 1},
        compiler_params=pltpu.CompilerParams(
            dimension_semantics=("arbitrary",),
        ),
    )(p_flat.reshape(-1, 128), i_flat.reshape(-1, 128), p_acc, i_acc)


def kernel(x, peso):
    s0 = _scores(x, peso, 0, _TOKENS // _BT)
    p0, i0 = _topk_sc(s0.reshape(-1), _TOKENS)
    pesos_t, idx_t = _pack_single(p0, i0)
    return jnp.transpose(pesos_t, (1, 0)), jnp.transpose(idx_t, (1, 0))


def _pack_single(p_flat, i_flat):
    return pl.pallas_call(
        _pack_body,
        grid=(_TOKENS // 2048,),
        in_specs=[
            pl.BlockSpec((128, 128), lambda i: (i, 0)),
            pl.BlockSpec((128, 128), lambda i: (i, 0)),
        ],
        out_specs=[
            pl.BlockSpec((_K, 2048), lambda i: (0, i)),
            pl.BlockSpec((_K, 2048), lambda i: (0, i)),
        ],
        out_shape=[
            jax.ShapeDtypeStruct((_K, _TOKENS), jnp.float32),
            jax.ShapeDtypeStruct((_K, _TOKENS), jnp.int32),
        ],
        compiler_params=pltpu.CompilerParams(
            dimension_semantics=("arbitrary",),
        ),
    )(p_flat.reshape(-1, 128), i_flat.reshape(-1, 128))


# speedup vs baseline: 1.0468x; 1.0468x over previous
import jax
import jax.numpy as jnp
from jax import lax
from jax.experimental import pallas as pl
from jax.experimental.pallas import tpu as pltpu
from jax.experimental.pallas import tpu_sc as plsc

_TOKENS = 32768
_DIM = 2048
_NE = 64
_K = 8
_BT = 2048
_NW = 32
_SPAN = 256
_CUT = 24576


def _score_body(x_ref, w_ref, s_ref):
    logits = lax.dot_general(
        x_ref[...], w_ref[...], (((1,), (1,)), ((), ())),
        preferred_element_type=jnp.float32,
    )
    m = jnp.max(logits, axis=1, keepdims=True)
    e = jnp.exp(logits - m)
    s = e / jnp.sum(e, axis=1, keepdims=True)
    parts = []
    for j in range(_BT // _SPAN):
        a = s[j * _SPAN: j * _SPAN + 128]
        b = s[j * _SPAN + 128: (j + 1) * _SPAN]
        parts.append(jnp.concatenate([a, b], axis=1))
    s_ref[...] = jnp.concatenate(parts, axis=0)


def _scores(x, peso, blk0, nblk):
    return pl.pallas_call(
        _score_body,
        grid=(nblk,),
        in_specs=[
            pl.BlockSpec((_BT, _DIM), lambda i: (blk0 + i, 0)),
            pl.BlockSpec((_NE, _DIM), lambda i: (0, 0)),
        ],
        out_specs=pl.BlockSpec((_BT // 2, 2 * _NE), lambda i: (i, 0)),
        out_shape=jax.ShapeDtypeStruct((nblk * _BT // 2, 2 * _NE), jnp.float32),
        compiler_params=pltpu.CompilerParams(
            dimension_semantics=("arbitrary",),
        ),
    )(x, peso)


def _merge(aK, aV, bK, bV):
    brK = lax.rev(bK, (0,))
    brV = lax.rev(bV, (0,))
    m = aK >= brK
    K = jnp.where(m, aK, brK)
    V = jnp.where(m, aV, brV)
    return plsc.sort_key_val(K, V, descending=True)


def _topk_sc(scores_flat, ntok):
    rpw = ntok // _NW

    def body(s_hbm, pesos_hbm, idx_hbm, s_v, p_v, i_v):
        wid = lax.axis_index("s") * 2 + lax.axis_index("c")
        base = wid * rpw
        pltpu.sync_copy(s_hbm.at[pl.ds(base * _NE, rpw * _NE)], s_v)

        lane = lax.iota(jnp.int32, 16)
        mask8 = lane < 8
        v0 = lane
        v1 = lane + 16
        v2 = lane + 32
        v3 = lane + 48
        lane128 = lane * 128

        @plsc.parallel_loop(0, rpw, unroll=8)
        def _row(r):
            off = (r // _SPAN) * (_SPAN * _NE) \
                + (r % 128) * 128 + ((r // 128) % 2) * _NE
            k0, i0 = plsc.sort_key_val(s_v[pl.ds(off, 16)], v0,
                                       descending=True)
            k1, i1 = plsc.sort_key_val(s_v[pl.ds(off + 16, 16)], v1,
                                       descending=True)
            k2, i2 = plsc.sort_key_val(s_v[pl.ds(off + 32, 16)], v2,
                                       descending=True)
            k3, i3 = plsc.sort_key_val(s_v[pl.ds(off + 48, 16)], v3,
                                       descending=True)
            ka, ia = _merge(k0, i0, k1, i1)
            kb, ib = _merge(k2, i2, k3, i3)
            kf, jf = _merge(ka, ia, kb, ib)
            pos = (r // 128) * 1024 + (r % 128) + lane128
            plsc.store_scatter(p_v, [pos], kf, mask=mask8)
            plsc.store_scatter(i_v, [pos], jf, mask=mask8)

        pltpu.sync_copy(p_v, pesos_hbm.at[pl.ds(base * _K, rpw * _K)])
        pltpu.sync_copy(i_v, idx_hbm.at[pl.ds(base * _K, rpw * _K)])

    mesh = plsc.VectorSubcoreMesh(core_axis_name="c", subcore_axis_name="s")
    f = pl.kernel(
        body,
        out_type=[
            jax.ShapeDtypeStruct((ntok * _K,), jnp.float32),
            jax.ShapeDtypeStruct((ntok * _K,), jnp.int32),
        ],
        mesh=mesh,
        scratch_types=[
            pltpu.VMEM((rpw * _NE,), jnp.float32),
            pltpu.VMEM((rpw * _K,), jnp.float32),
            pltpu.VMEM((rpw * _K,), jnp.int32),
        ],
        compiler_params=pltpu.CompilerParams(needs_layout_passes=False),
    )
    return f(scores_flat)


def _pack_body(pf_ref, if_ref, p_ref, i_ref):
    ng = pf_ref.shape[0] // 8
    a3 = pf_ref[...].reshape(ng, 8, 128)
    b3 = if_ref[...].reshape(ng, 8, 128)
    for j in range(ng):
        p_ref[:, pl.ds(128 * j, 128)] = a3[j]
        i_ref[:, pl.ds(128 * j, 128)] = b3[j]


def _pack_body2(pf_ref, if_ref, pa_ref, ia_ref, p_ref, i_ref):
    del pa_ref, ia_ref
    _pack_body(pf_ref, if_ref, p_ref, i_ref)


def _pack0(p_flat, i_flat):
    n = _CUT // 2048
    return pl.pallas_call(
        _pack_body,
        grid=(n,),
        in_specs=[
            pl.BlockSpec((128, 128), lambda i: (i, 0)),
            pl.BlockSpec((128, 128), lambda i: (i, 0)),
        ],
        out_specs=[
            pl.BlockSpec((_K, 2048), lambda i: (0, i)),
            pl.BlockSpec((_K, 2048), lambda i: (0, i)),
        ],
        out_shape=[
            jax.ShapeDtypeStruct((_K, _TOKENS), jnp.float32),
            jax.ShapeDtypeStruct((_K, _TOKENS), jnp.int32),
        ],
        compiler_params=pltpu.CompilerParams(
            dimension_semantics=("arbitrary",),
        ),
    )(p_flat.reshape(-1, 128), i_flat.reshape(-1, 128))


def _pack1(p_flat, i_flat, p_acc, i_acc):
    base = _CUT // 2048
    n = (_TOKENS - _CUT) // 2048
    return pl.pallas_call(
        _pack_body2,
        grid=(n,),
        in_specs=[
            pl.BlockSpec((128, 128), lambda i: (i, 0)),
            pl.BlockSpec((128, 128), lambda i: (i, 0)),
            pl.BlockSpec(memory_space=pl.ANY),
            pl.BlockSpec(memory_space=pl.ANY),
        ],
        out_specs=[
            pl.BlockSpec((_K, 2048), lambda i: (0, base + i)),
            pl.BlockSpec((_K, 2048), lambda i: (0, base + i)),
        ],
        out_shape=[
            jax.ShapeDtypeStruct((_K, _TOKENS), jnp.float32),
            jax.ShapeDtypeStruct((_K, _TOKENS), jnp.int32),
        ],
        input_output_aliases={2: 0, 3: 1},
        compiler_params=pltpu.CompilerParams(
            dimension_semantics=("arbitrary",),
        ),
    )(p_flat.reshape(-1, 128), i_flat.reshape(-1, 128), p_acc, i_acc)


def kernel(x, peso):
    s0 = _scores(x, peso, 0, _TOKENS // _BT)
    p0, i0 = _topk_sc(s0.reshape(-1), _TOKENS)
    pesos_t, idx_t = _pack_single(p0, i0)
    return jnp.transpose(pesos_t, (1, 0)), jnp.transpose(idx_t, (1, 0))


def _pack_single(p_flat, i_flat):
    return pl.pallas_call(
        _pack_body,
        grid=(4,),
        in_specs=[
            pl.BlockSpec((512, 128), lambda i: (i, 0)),
            pl.BlockSpec((512, 128), lambda i: (i, 0)),
        ],
        out_specs=[
            pl.BlockSpec((_K, 8192), lambda i: (0, i)),
            pl.BlockSpec((_K, 8192), lambda i: (0, i)),
        ],
        out_shape=[
            jax.ShapeDtypeStruct((_K, _TOKENS), jnp.float32),
            jax.ShapeDtypeStruct((_K, _TOKENS), jnp.int32),
        ],
        compiler_params=pltpu.CompilerParams(
            dimension_semantics=("arbitrary",),
        ),
    )(p_flat.reshape(-1, 128), i_flat.reshape(-1, 128))


# speedup vs baseline: 1.0657x; 1.0180x over previous
import jax
import jax.numpy as jnp
from jax import lax
from jax.experimental import pallas as pl
from jax.experimental.pallas import tpu as pltpu
from jax.experimental.pallas import tpu_sc as plsc

_TOKENS = 32768
_DIM = 2048
_NE = 64
_K = 8
_BT = 2048
_NW = 32
_SPAN = 256
_CUT = 24576


def _score_body(x_ref, w_ref, s_ref):
    logits = lax.dot_general(
        x_ref[...], w_ref[...], (((1,), (1,)), ((), ())),
        preferred_element_type=jnp.float32,
    )
    m = jnp.max(logits, axis=1, keepdims=True)
    e = jnp.exp(logits - m)
    s = e / jnp.sum(e, axis=1, keepdims=True)
    parts = []
    for j in range(_BT // _SPAN):
        a = s[j * _SPAN: j * _SPAN + 128]
        b = s[j * _SPAN + 128: (j + 1) * _SPAN]
        parts.append(jnp.concatenate([a, b], axis=1))
    s_ref[...] = jnp.concatenate(parts, axis=0)


def _scores(x, peso, blk0, nblk):
    return pl.pallas_call(
        _score_body,
        grid=(nblk,),
        in_specs=[
            pl.BlockSpec((_BT, _DIM), lambda i: (blk0 + i, 0)),
            pl.BlockSpec((_NE, _DIM), lambda i: (0, 0)),
        ],
        out_specs=pl.BlockSpec((_BT // 2, 2 * _NE), lambda i: (i, 0)),
        out_shape=jax.ShapeDtypeStruct((nblk * _BT // 2, 2 * _NE), jnp.float32),
        compiler_params=pltpu.CompilerParams(
            dimension_semantics=("arbitrary",),
        ),
    )(x, peso)


def _merge(aK, aV, bK, bV):
    brK = lax.rev(bK, (0,))
    brV = lax.rev(bV, (0,))
    m = aK >= brK
    K = jnp.where(m, aK, brK)
    V = jnp.where(m, aV, brV)
    return plsc.sort_key_val(K, V, descending=True)


def _topk_sc(scores_flat, ntok):
    rpw = ntok // _NW

    def body(s_hbm, pesos_hbm, idx_hbm, s_v, p_v, i_v):
        wid = lax.axis_index("s") * 2 + lax.axis_index("c")
        base = wid * rpw
        pltpu.sync_copy(s_hbm.at[pl.ds(base * _NE, rpw * _NE)], s_v)

        lane = lax.iota(jnp.int32, 16)
        mask8 = lane < 8
        v0 = lane
        v1 = lane + 16
        v2 = lane + 32
        v3 = lane + 48
        lane128 = lane * 128

        @plsc.parallel_loop(0, rpw, unroll=8)
        def _row(r):
            off = (r // _SPAN) * (_SPAN * _NE) \
                + (r % 128) * 128 + ((r // 128) % 2) * _NE
            k0, i0 = plsc.sort_key_val(s_v[pl.ds(off, 16)], v0,
                                       descending=True)
            k1, i1 = plsc.sort_key_val(s_v[pl.ds(off + 16, 16)], v1,
                                       descending=True)
            k2, i2 = plsc.sort_key_val(s_v[pl.ds(off + 32, 16)], v2,
                                       descending=True)
            k3, i3 = plsc.sort_key_val(s_v[pl.ds(off + 48, 16)], v3,
                                       descending=True)
            ka, ia = _merge(k0, i0, k1, i1)
            kb, ib = _merge(k2, i2, k3, i3)
            kf, jf = _merge(ka, ia, kb, ib)
            pos = (r // 128) * 1024 + (r % 128) + lane128
            plsc.store_scatter(p_v, [pos], kf, mask=mask8)
            plsc.store_scatter(i_v, [pos], jf, mask=mask8)

        pltpu.sync_copy(p_v, pesos_hbm.at[pl.ds(base * _K, rpw * _K)])
        pltpu.sync_copy(i_v, idx_hbm.at[pl.ds(base * _K, rpw * _K)])

    mesh = plsc.VectorSubcoreMesh(core_axis_name="c", subcore_axis_name="s")
    f = pl.kernel(
        body,
        out_type=[
            jax.ShapeDtypeStruct((ntok * _K,), jnp.float32),
            jax.ShapeDtypeStruct((ntok * _K,), jnp.int32),
        ],
        mesh=mesh,
        scratch_types=[
            pltpu.VMEM((rpw * _NE,), jnp.float32),
            pltpu.VMEM((rpw * _K,), jnp.float32),
            pltpu.VMEM((rpw * _K,), jnp.int32),
        ],
        compiler_params=pltpu.CompilerParams(needs_layout_passes=False),
    )
    return f(scores_flat)


def _pack_body(pf_ref, if_ref, p_ref, i_ref):
    ng = pf_ref.shape[0] // 8
    a3 = pf_ref[...].reshape(ng, 8, 128)
    b3 = if_ref[...].reshape(ng, 8, 128)
    for j in range(ng):
        p_ref[:, pl.ds(128 * j, 128)] = a3[j]
        i_ref[:, pl.ds(128 * j, 128)] = b3[j]


def _pack_body2(pf_ref, if_ref, pa_ref, ia_ref, p_ref, i_ref):
    del pa_ref, ia_ref
    _pack_body(pf_ref, if_ref, p_ref, i_ref)


def _pack0(p_flat, i_flat):
    n = _CUT // 8192
    return pl.pallas_call(
        _pack_body,
        grid=(n,),
        in_specs=[
            pl.BlockSpec((512, 128), lambda i: (i, 0)),
            pl.BlockSpec((512, 128), lambda i: (i, 0)),
        ],
        out_specs=[
            pl.BlockSpec((_K, 8192), lambda i: (0, i)),
            pl.BlockSpec((_K, 8192), lambda i: (0, i)),
        ],
        out_shape=[
            jax.ShapeDtypeStruct((_K, _TOKENS), jnp.float32),
            jax.ShapeDtypeStruct((_K, _TOKENS), jnp.int32),
        ],
        compiler_params=pltpu.CompilerParams(
            dimension_semantics=("arbitrary",),
        ),
    )(p_flat.reshape(-1, 128), i_flat.reshape(-1, 128))


def _pack1(p_flat, i_flat, p_acc, i_acc):
    base = _CUT // 8192
    n = (_TOKENS - _CUT) // 8192
    return pl.pallas_call(
        _pack_body2,
        grid=(n,),
        in_specs=[
            pl.BlockSpec((512, 128), lambda i: (i, 0)),
            pl.BlockSpec((512, 128), lambda i: (i, 0)),
            pl.BlockSpec(memory_space=pl.ANY),
            pl.BlockSpec(memory_space=pl.ANY),
        ],
        out_specs=[
            pl.BlockSpec((_K, 8192), lambda i: (0, base + i)),
            pl.BlockSpec((_K, 8192), lambda i: (0, base + i)),
        ],
        out_shape=[
            jax.ShapeDtypeStruct((_K, _TOKENS), jnp.float32),
            jax.ShapeDtypeStruct((_K, _TOKENS), jnp.int32),
        ],
        input_output_aliases={2: 0, 3: 1},
        compiler_params=pltpu.CompilerParams(
            dimension_semantics=("arbitrary",),
        ),
    )(p_flat.reshape(-1, 128), i_flat.reshape(-1, 128), p_acc, i_acc)


def kernel(x, peso):
    s0 = _scores(x, peso, 0, _CUT // _BT)
    p0, i0 = _topk_sc(s0.reshape(-1), _CUT)
    s1 = _scores(x, peso, _CUT // _BT, (_TOKENS - _CUT) // _BT)
    p1, i1 = _topk_sc(s1.reshape(-1), _TOKENS - _CUT)
    pa, ia = _pack0(p0, i0)
    pesos_t, idx_t = _pack1(p1, i1, pa, ia)
    return jnp.transpose(pesos_t, (1, 0)), jnp.transpose(idx_t, (1, 0))


def _pack_single(p_flat, i_flat):
    return pl.pallas_call(
        _pack_body,
        grid=(4,),
        in_specs=[
            pl.BlockSpec((512, 128), lambda i: (i, 0)),
            pl.BlockSpec((512, 128), lambda i: (i, 0)),
        ],
        out_specs=[
            pl.BlockSpec((_K, 8192), lambda i: (0, i)),
            pl.BlockSpec((_K, 8192), lambda i: (0, i)),
        ],
        out_shape=[
            jax.ShapeDtypeStruct((_K, _TOKENS), jnp.float32),
            jax.ShapeDtypeStruct((_K, _TOKENS), jnp.int32),
        ],
        compiler_params=pltpu.CompilerParams(
            dimension_semantics=("arbitrary",),
        ),
    )(p_flat.reshape(-1, 128), i_flat.reshape(-1, 128))
